# baseline (speedup 1.0000x reference)
import functools

import jax
import jax.numpy as jnp
from jax import lax
from jax.experimental import pallas as pl
from jax.experimental.pallas import tpu as pltpu
from jax.experimental.pallas import tpu_sc as plsc

NP = 5120
M = 256
NW = 16
RPW = NP // NW
NG = RPW // 16
L = 16
NCH = M // L
BIGI = 0x3FFFFFFF


def _init_kernel(xr_ref, yr_ref, gxc_ref, gyc_ref, objc_ref, oidr_ref,
                 tobjc_ref, tgtc_ref, dist_out, gt0_out, obj0_out,
                 rowfree_out, colfree_out, score_out):
    Mm = oidr_ref.shape[0]
    Nn = tobjc_ref.shape[1]
    dist_out[...] = ((xr_ref[...] - gxc_ref[...]) ** 2
                     + (yr_ref[...] - gyc_ref[...]) ** 2)
    tobj = tobjc_ref[...]
    oid = oidr_ref[...]
    match = tobj == oid
    has_match = jnp.any(match, axis=0, keepdims=True)
    iota_m = lax.broadcasted_iota(jnp.int32, (Mm, Nn), 0)
    jj_first = jnp.min(jnp.where(match, iota_m, BIGI), axis=0, keepdims=True)
    gt0_out[...] = jnp.where(has_match, jj_first, -1).astype(jnp.int32)
    obj0_out[...] = tobj
    rowfree_out[...] = (~((tgtc_ref[...] >= 0) | has_match)).astype(jnp.int32)
    colfree_out[...] = (~jnp.any(match, axis=1, keepdims=True)).astype(jnp.int32)
    score_out[...] = jax.nn.sigmoid(objc_ref[...])


def _sc_greedy(dist_hbm, gt0_hbm, obj0_hbm, rowfree_hbm, colfree_hbm,
               oid_hbm, gt_out, obj_out,
               dist_v, gt0_v, obj0_v, rowfree_v, pen_v, oid_v,
               bufv_v, bufi_v, bufc_v, outg_v, outo_v,
               rowfree_s, rowarg_s, garg_s, claim_s, go_s,
               sh_val, sh_idx, sh_claim):
    wid = lax.axis_index("s")
    base = wid * RPW
    inf = jnp.float32(jnp.inf)
    lane = lax.broadcasted_iota(jnp.int32, (L,), 0)
    perms = tuple(lane ^ k for k in (8, 4, 2, 1))

    def bmin(a):
        for p in perms:
            a = jnp.minimum(a, a.at[p].get(mode="promise_in_bounds"))
        return a

    def bmax(a):
        for p in perms:
            a = jnp.maximum(a, a.at[p].get(mode="promise_in_bounds"))
        return a

    pltpu.sync_copy(dist_hbm.at[pl.ds(base, RPW)], dist_v)
    pltpu.sync_copy(gt0_hbm.at[pl.ds(base, RPW)], gt0_v)
    pltpu.sync_copy(obj0_hbm.at[pl.ds(base, RPW)], obj0_v)
    pltpu.sync_copy(rowfree_hbm.at[pl.ds(base, RPW)], rowfree_v)
    pltpu.sync_copy(colfree_hbm, bufi_v.at[0])
    pltpu.sync_copy(oid_hbm, oid_v)

    def pen_init(c, d):
        cf = bufi_v[0, pl.ds(c * L, L)]
        pen_v[pl.ds(c * L, L)] = jnp.where(
            cf != 0, jnp.zeros((L,), jnp.float32), inf)
        return d

    lax.fori_loop(0, NCH, pen_init, jnp.int32(0))

    def rf_init(g, d):
        rf = rowfree_v[pl.ds(g * L, L)]
        for p in range(L):
            rowfree_s[g * L + p] = rf[p]
        return d

    lax.fori_loop(0, NG, rf_init, jnp.int32(0))
    go_s[0] = jnp.int32(1)

    def round_payload():
        def part_init(c, d):
            bufv_v[0, pl.ds(c * L, L)] = jnp.full((L,), inf)
            bufi_v[0, pl.ds(c * L, L)] = jnp.full((L,), BIGI, jnp.int32)
            return d

        lax.fori_loop(0, NCH, part_init, jnp.int32(0))
        pens = tuple(pen_v[pl.ds(c * L, L)] for c in range(NCH))

        def row_step(i, dummy):
            @pl.when(rowfree_s[i] == 1)
            def _():
                rowg = jnp.full((L,), base, jnp.int32) + i
                rm = jnp.full((L,), inf)
                ac = jnp.zeros((L,), jnp.int32)
                for c in range(NCH):
                    d = dist_v[i, pl.ds(c * L, L)]
                    m = d + pens[c]
                    lt = m < rm
                    rm = jnp.where(lt, m, rm)
                    ac = jnp.where(lt, c, ac)
                    cmc = bufv_v[0, pl.ds(c * L, L)]
                    lt2 = m < cmc
                    cac = bufi_v[0, pl.ds(c * L, L)]
                    bufv_v[0, pl.ds(c * L, L)] = jnp.where(lt2, m, cmc)
                    bufi_v[0, pl.ds(c * L, L)] = jnp.where(lt2, rowg, cac)
                mv = bmin(rm)
                cand = jnp.where(rm == mv, ac * L + lane, BIGI)
                cb = bmin(cand)
                rowarg_s[i] = jnp.where(mv[0] < inf, cb[0], BIGI)
            return dummy

        lax.fori_loop(0, RPW, row_step, jnp.int32(0))

        pltpu.sync_copy(bufv_v.at[0], sh_val.at[wid])
        pltpu.sync_copy(bufi_v.at[0], sh_idx.at[wid])
        plsc.subcore_barrier()
        pltpu.sync_copy(sh_val, bufv_v)
        pltpu.sync_copy(sh_idx, bufi_v)
        def merge_step(c, d):
            gm = jnp.full((L,), inf)
            ga = jnp.full((L,), BIGI, jnp.int32)
            for w in range(NW):
                v = bufv_v[w, pl.ds(c * L, L)]
                ix = bufi_v[w, pl.ds(c * L, L)]
                lt = v < gm
                gm = jnp.where(lt, v, gm)
                ga = jnp.where(lt, ix, ga)
            for p in range(L):
                garg_s[c * L + p] = ga[p]
            return d

        lax.fori_loop(0, NCH, merge_step, jnp.int32(0))

        def claim_zero(c, d):
            claim_s[c] = jnp.int32(0)
            return d

        lax.fori_loop(0, M, claim_zero, jnp.int32(0))

        def mutual_step(i, dummy):
            @pl.when(rowfree_s[i] == 1)
            def _():
                j = rowarg_s[i]

                @pl.when(j < BIGI)
                def _():
                    @pl.when(garg_s[j] == base + i)
                    def _():
                        rowfree_s[i] = jnp.int32(2)
                        claim_s[j] = jnp.int32(1)
            return dummy

        lax.fori_loop(0, RPW, mutual_step, jnp.int32(0))

        def claim_pub(c, d):
            acc = jnp.zeros((L,), jnp.int32)
            for p in range(L):
                acc = jnp.where(lane == p, claim_s[c * L + p], acc)
            bufc_v[0, pl.ds(c * L, L)] = acc
            return d

        lax.fori_loop(0, NCH, claim_pub, jnp.int32(0))
        pltpu.sync_copy(bufc_v.at[0], sh_claim.at[wid])
        plsc.subcore_barrier()
        pltpu.sync_copy(sh_claim, bufc_v)

        def claim_merge(c, d):
            cl = jnp.zeros((L,), jnp.int32)
            for w in range(NW):
                cl = cl | bufc_v[w, pl.ds(c * L, L)]
            p = pen_v[pl.ds(c * L, L)]
            pen_v[pl.ds(c * L, L)] = jnp.where(cl != 0, inf, p)
            return d | bmax(cl)[0]

        go_s[0] = lax.fori_loop(0, NCH, claim_merge, jnp.int32(0))
        plsc.subcore_barrier()

    def round_body(r, dummy):
        @pl.when(go_s[0] > 0)
        def _():
            round_payload()
        return dummy

    lax.fori_loop(0, M + 2, round_body, jnp.int32(0))

    def final_step(g, d):
        won = jnp.zeros((L,), jnp.int32)
        gtw = jnp.zeros((L,), jnp.int32)
        for p in range(L):
            i = g * L + p
            w = rowfree_s[i] == 2
            jw = jnp.where(w, rowarg_s[i], 0)
            won = jnp.where(lane == p, jnp.where(w, 1, 0), won)
            gtw = jnp.where(lane == p, jw, gtw)
        wonb = won != 0
        outg_v[pl.ds(g * L, L)] = jnp.where(wonb, gtw, gt0_v[pl.ds(g * L, L)])
        outo_v[pl.ds(g * L, L)] = jnp.where(wonb, gtw, obj0_v[pl.ds(g * L, L)])
        return d

    lax.fori_loop(0, NG, final_step, jnp.int32(0))
    pltpu.sync_copy(outg_v, gt_out.at[pl.ds(base, RPW)])
    pltpu.sync_copy(outo_v, obj_out.at[pl.ds(base, RPW)])


def kernel(is_object, boxes, gt_boxes, obj_ids, track_obj_idx, track_gt_idx):
    N = track_obj_idx.shape[0]
    pad = NP - N
    x = jnp.pad(boxes[-1, 0, :, 0], (0, pad))
    y = jnp.pad(boxes[-1, 0, :, 1], (0, pad))
    obj = jnp.pad(is_object[-1, 0, :, 0], (0, pad))
    tobj = jnp.pad(track_obj_idx, (0, pad), constant_values=-1)
    tgt = jnp.pad(track_gt_idx, (0, pad), constant_values=0)

    dist, gt0, obj0, rowfree, colfree, score = pl.pallas_call(
        _init_kernel,
        out_shape=(
            jax.ShapeDtypeStruct((NP, M), jnp.float32),
            jax.ShapeDtypeStruct((1, NP), jnp.int32),
            jax.ShapeDtypeStruct((1, NP), jnp.int32),
            jax.ShapeDtypeStruct((1, NP), jnp.int32),
            jax.ShapeDtypeStruct((M, 1), jnp.int32),
            jax.ShapeDtypeStruct((1, NP), jnp.float32),
        ),
    )(x.reshape(NP, 1), y.reshape(NP, 1), gt_boxes[:, 0].reshape(1, M),
      gt_boxes[:, 1].reshape(1, M), obj.reshape(1, NP),
      obj_ids.reshape(M, 1), tobj.reshape(1, NP), tgt.reshape(1, NP))

    mesh = plsc.VectorSubcoreMesh(core_axis_name="c", subcore_axis_name="s",
                                  num_cores=1, num_subcores=NW)
    sc = functools.partial(
        pl.kernel,
        mesh=mesh,
        out_type=(
            jax.ShapeDtypeStruct((NP,), jnp.int32),
            jax.ShapeDtypeStruct((NP,), jnp.int32),
        ),
        scratch_types=[
            pltpu.VMEM((RPW, M), jnp.float32),
            pltpu.VMEM((RPW,), jnp.int32),
            pltpu.VMEM((RPW,), jnp.int32),
            pltpu.VMEM((RPW,), jnp.int32),
            pltpu.VMEM((M,), jnp.float32),
            pltpu.VMEM((M,), jnp.int32),
            pltpu.VMEM((NW, M), jnp.float32),
            pltpu.VMEM((NW, M), jnp.int32),
            pltpu.VMEM((NW, M), jnp.int32),
            pltpu.VMEM((RPW,), jnp.int32),
            pltpu.VMEM((RPW,), jnp.int32),
            pltpu.SMEM((RPW,), jnp.int32),
            pltpu.SMEM((RPW,), jnp.int32),
            pltpu.SMEM((M,), jnp.int32),
            pltpu.SMEM((M,), jnp.int32),
            pltpu.SMEM((1,), jnp.int32),
            pltpu.VMEM_SHARED((NW, M), jnp.float32),
            pltpu.VMEM_SHARED((NW, M), jnp.int32),
            pltpu.VMEM_SHARED((NW, M), jnp.int32),
        ],
    )(_sc_greedy)
    gt, ob = sc(dist, gt0.reshape(NP), obj0.reshape(NP), rowfree.reshape(NP),
                colfree.reshape(M), obj_ids)
    return gt[:N], ob[:N], score.reshape(NP)[:N]
